# baseline (speedup 1.0000x reference)
import functools

import jax
import jax.numpy as jnp
from jax import lax
from jax.experimental import pallas as pl
from jax.experimental.pallas import tpu as pltpu
from jax.experimental.pallas import tpu_sc as plsc

NC = 2
NS = 16
L = 16
NW = NC * NS

BATCH_ROWS = 16384
E_ROWS = 1000000
R_ROWS = 1000
DIM = 64
HD = DIM // 2
DIMS_PER_SC = HD // NC
ROWS_PER_TEC = BATCH_ROWS // NS
CHUNK = 128
NCH = ROWS_PER_TEC // CHUNK
CPOS = CHUNK // L

BE = 2048
PACK_ROWS = E_ROWS // 4
HROWS = BATCH_ROWS // NW
HCH = HROWS // CHUNK


def _sqrt16(x):
    xs = jnp.maximum(x, jnp.float32(1e-30))
    i = lax.bitcast_convert_type(xs, jnp.int32)
    y = lax.bitcast_convert_type(jnp.int32(0x5F3759DF) - (i >> 1), jnp.float32)
    for _ in range(3):
        y = y * (jnp.float32(1.5) - jnp.float32(0.5) * xs * y * y)
    return xs * y


def _transpose_body(eT_ref, out_ref):
    t = eT_ref[...].T
    Q = BE // 4
    out_ref[...] = jnp.concatenate(
        [t[0:Q], t[Q:2 * Q], t[2 * Q:3 * Q], t[3 * Q:4 * Q]], axis=1)


def _accum_body(train_r_hbm, p_h_hbm, p_t_hbm, n_h_hbm, n_t_hbm,
                eT_hbm, rT_hbm, out_hbm,
                ir0, ir1, iph0, iph1, ipt0, ipt1, inh0, inh1, int0, int1,
                wr0, wr1, wph0, wph1, wpt0, wpt1, wnh0, wnh1, wnt0, wnt1,
                sq_p, sq_n,
                rrow0, rrow1, stripe0, stripe1,
                sem_s, sem_r, sem_i, sem_g):
    cid = lax.axis_index("c")
    sid = lax.axis_index("s")
    base = sid * ROWS_PER_TEC
    dim0 = cid * DIMS_PER_SC

    idxb = ((ir0, iph0, ipt0, inh0, int0), (ir1, iph1, ipt1, inh1, int1))
    wb = ((wr0, wph0, wpt0, wnh0, wnt0), (wr1, wph1, wpt1, wnh1, wnt1))
    stripes = (stripe0, stripe1)
    rrows = (rrow0, rrow1)
    src_idx = (train_r_hbm, p_h_hbm, p_t_hbm, n_h_hbm, n_t_hbm)

    zv = jnp.zeros((L,), jnp.float32)

    def zero_body(i, carry):
        sl = pl.ds(i * L, L)
        sq_p[sl] = zv
        sq_n[sl] = zv
        return carry

    lax.fori_loop(0, ROWS_PER_TEC // L, zero_body, jnp.int32(0))

    def fire_idx(cc, par):
        off = base + cc * CHUNK
        for hsrc, dst in zip(src_idx, idxb[par]):
            pltpu.async_copy(hsrc.at[pl.ds(off, CHUNK)], dst, sem_i)

    def drain_idx(par):
        for dst in idxb[par]:
            pltpu.make_async_copy(train_r_hbm.at[pl.ds(0, CHUNK)],
                                  dst, sem_i).wait()

    def fire_gather(par, stripe, rrowb):
        ib = idxb[par]
        ob = wb[par]
        pltpu.async_copy(rrowb.at[ib[0]], ob[0], sem_g)
        for k in range(1, 5):
            pltpu.async_copy(stripe.at[ib[k]], ob[k], sem_g)

    def drain_gather(par):
        for dst in wb[par]:
            pltpu.make_async_copy(out_hbm.at[0, 0, pl.ds(0, CHUNK)],
                                  dst, sem_g).wait()

    def compute(cc, par):
        vr, vph, vpt, vnh, vnt = wb[par]

        def pos_body(i, carry):
            sl = pl.ds(i * L, L)
            gl = pl.ds(cc * CHUNK + i * L, L)
            rv = vr[sl]
            dp = vph[sl] + rv - vpt[sl]
            dn = vnh[sl] + rv - vnt[sl]
            sq_p[gl] = sq_p[gl] + dp * dp
            sq_n[gl] = sq_n[gl] + dn * dn
            return carry

        lax.fori_loop(0, CPOS, pos_body, jnp.int32(0))

    def process(sp, rp):
        stripe, rrowb = stripes[sp], rrows[rp]

        def chunk_body(ii, carry):
            for v in range(2):
                cc = ii * 2 + v

                @pl.when(cc >= 2)
                def _():
                    drain_gather(v)
                    compute(cc - 2, v)

                @pl.when(cc < NCH)
                def _():
                    fire_idx(cc, v)

                @pl.when((cc >= 1) & (cc < NCH + 1))
                def _():
                    drain_idx(1 - v)
                    fire_gather(1 - v, stripe, rrowb)
            return carry

        lax.fori_loop(0, NCH // 2 + 1, chunk_body, jnp.int32(0))

    def dim_body(jj, carry):
        for u in range(2):
            j = jj * 2 + u

            @pl.when((j > 0) & (j <= DIMS_PER_SC) & (sid == (j - 1) % NS))
            def _():
                pltpu.make_async_copy(eT_hbm.at[0],
                                      stripes[1 - u], sem_s).wait()

            @pl.when((j > 0) & (j <= DIMS_PER_SC) & (sid == (j + 7) % NS))
            def _():
                pltpu.make_async_copy(rT_hbm.at[0],
                                      rrows[1 - u], sem_r).wait()

            plsc.subcore_barrier()
            d = dim0 + j

            @pl.when((j < DIMS_PER_SC) & (sid == j % NS))
            def _():
                pltpu.async_copy(eT_hbm.at[d], stripes[u], sem_s)

            @pl.when((j < DIMS_PER_SC) & (sid == (j + 8) % NS))
            def _():
                pltpu.async_copy(rT_hbm.at[d], rrows[u], sem_r)

            @pl.when((j > 0) & (j <= DIMS_PER_SC))
            def _():
                process(1 - u, 1 - u)
        return carry

    lax.fori_loop(0, DIMS_PER_SC // 2 + 1, dim_body, jnp.int32(0))

    pltpu.sync_copy(sq_p, out_hbm.at[cid, 0, pl.ds(base, ROWS_PER_TEC)])
    pltpu.sync_copy(sq_n, out_hbm.at[cid, 1, pl.ds(base, ROWS_PER_TEC)])


def _half_body(train_r_hbm, p_h_hbm, p_t_hbm, n_h_hbm, n_t_hbm,
               packed_hbm, packed_r_hbm, out_hbm,
               iraw, itr_r, itr_ph, itr_pt, itr_nh, itr_nt,
               iq_r, iq_ph, iq_pt, iq_nh, iq_nt,
               gr, gph, gpt, gnh, gnt,
               sq_p, sq_n, sem_g):
    cid = lax.axis_index("c")
    sid = lax.axis_index("s")
    wid = sid * NC + cid
    base = wid * HROWS
    iota16 = lax.iota(jnp.int32, L)

    src_idx = (train_r_hbm, p_h_hbm, p_t_hbm, n_h_hbm, n_t_hbm)
    itrs = (itr_r, itr_ph, itr_pt, itr_nh, itr_nt)
    iqs = (iq_r, iq_ph, iq_pt, iq_nh, iq_nt)
    gbufs = (gr, gph, gpt, gnh, gnt)

    def chunk_loop(cc, carry):
        off = base + cc * CHUNK
        for s in range(5):
            pltpu.sync_copy(src_idx[s].at[pl.ds(off, CHUNK)], iraw)

            def trans_body(i, carry2):
                sl = pl.ds(i * L, L)
                e = iraw[sl]
                if s == 0:
                    itrs[s][sl] = e & 255
                    iqs[s][sl] = (e >> 8) & 3
                else:
                    itrs[s][sl] = ((e >> 11) << 9) | (e & 511)
                    iqs[s][sl] = (e >> 9) & 3
                return carry2

            lax.fori_loop(0, CHUNK // L, trans_body, jnp.int32(0))
            src = packed_r_hbm if s == 0 else packed_hbm
            pltpu.async_copy(src.at[itrs[s]], gbufs[s], sem_g)
        for s in range(5):
            pltpu.make_async_copy(packed_hbm.at[pl.ds(0, CHUNK)],
                                  gbufs[s], sem_g).wait()

        def pos_body(i, carry2):
            sl = pl.ds(i * L, L)
            gsl = pl.ds(cc * CHUNK + i * L, L)
            rows = i * L + iota16
            br = iq_r[sl] * 32
            bph = iq_ph[sl] * 32
            bpt = iq_pt[sl] * 32
            bnh = iq_nh[sl] * 32
            bnt = iq_nt[sl] * 32

            def d_body(d, carry3):
                ap, an = carry3
                rv = plsc.load_gather(gr, [rows, br + d])
                dp = (plsc.load_gather(gph, [rows, bph + d]) + rv
                      - plsc.load_gather(gpt, [rows, bpt + d]))
                dn = (plsc.load_gather(gnh, [rows, bnh + d]) + rv
                      - plsc.load_gather(gnt, [rows, bnt + d]))
                return (ap + dp * dp, an + dn * dn)

            accp, accn = lax.fori_loop(
                0, HD, d_body,
                (jnp.zeros((L,), jnp.float32), jnp.zeros((L,), jnp.float32)))
            sq_p[gsl] = accp
            sq_n[gsl] = accn
            return carry2

        lax.fori_loop(0, CHUNK // L, pos_body, jnp.int32(0))
        return carry

    lax.fori_loop(0, HCH, chunk_loop, jnp.int32(0))

    pltpu.sync_copy(sq_p, out_hbm.at[0, pl.ds(base, HROWS)])
    pltpu.sync_copy(sq_n, out_hbm.at[1, pl.ds(base, HROWS)])


def _finish_body(part_hbm, half_hbm, out_hbm, v0, v1, v2, v3, h0, h1,
                 accbuf, sem):
    cid = lax.axis_index("c")
    sid = lax.axis_index("s")
    wid = sid * NC + cid
    base = wid * HROWS
    d0 = pltpu.async_copy(part_hbm.at[0, 0, pl.ds(base, HROWS)], v0, sem)
    d1 = pltpu.async_copy(part_hbm.at[1, 0, pl.ds(base, HROWS)], v1, sem)
    d2 = pltpu.async_copy(part_hbm.at[0, 1, pl.ds(base, HROWS)], v2, sem)
    d3 = pltpu.async_copy(part_hbm.at[1, 1, pl.ds(base, HROWS)], v3, sem)
    d4 = pltpu.async_copy(half_hbm.at[0, pl.ds(base, HROWS)], h0, sem)
    d5 = pltpu.async_copy(half_hbm.at[1, pl.ds(base, HROWS)], h1, sem)
    d0.wait(); d1.wait(); d2.wait(); d3.wait(); d4.wait(); d5.wait()
    acc = jnp.zeros((L,), jnp.float32)
    for i in range(HROWS // L):
        sl = pl.ds(i * L, L)
        acc = (acc + _sqrt16(v0[sl] + v1[sl] + h0[sl])
               - _sqrt16(v2[sl] + v3[sl] + h1[sl]))
    accbuf[...] = acc
    pltpu.sync_copy(accbuf, out_hbm.at[wid])


def kernel(train_r, p_h, p_t, n_h, n_t, e_embed, r_embed):
    mesh = plsc.VectorSubcoreMesh(
        core_axis_name="c", subcore_axis_name="s",
        num_cores=NC, num_subcores=NS)
    cp = pltpu.CompilerParams(needs_layout_passes=False)
    eT = e_embed.T
    rT = r_embed.T

    tc_pack = pl.pallas_call(
        _transpose_body,
        grid=(E_ROWS // BE,),
        in_specs=[pl.BlockSpec((HD, BE), lambda i: (1, i))],
        out_specs=pl.BlockSpec((BE // 4, 128), lambda i: (i, 0)),
        out_shape=jax.ShapeDtypeStruct((PACK_ROWS, 128), jnp.float32),
    )
    packed = tc_pack(eT)

    rpad = jnp.pad(r_embed[:, HD:], ((0, 1024 - R_ROWS), (0, 0)))
    packed_r = jnp.concatenate(
        [rpad[0:256], rpad[256:512], rpad[512:768], rpad[768:1024]], axis=1)

    k1 = functools.partial(
        pl.kernel,
        out_type=jax.ShapeDtypeStruct((NC, 2, BATCH_ROWS), jnp.float32),
        mesh=mesh,
        compiler_params=cp,
        scratch_types=(
            [pltpu.VMEM((CHUNK,), jnp.int32)] * 10
            + [pltpu.VMEM((CHUNK,), jnp.float32)] * 10
            + [pltpu.VMEM((ROWS_PER_TEC,), jnp.float32)] * 2
            + [pltpu.MemorySpace.VMEM_SHARED((R_ROWS,), jnp.float32)] * 2
            + [pltpu.MemorySpace.VMEM_SHARED((E_ROWS,), jnp.float32)] * 2
            + [pltpu.SemaphoreType.DMA] * 4
        ),
    )(_accum_body)

    kh = functools.partial(
        pl.kernel,
        out_type=jax.ShapeDtypeStruct((2, BATCH_ROWS), jnp.float32),
        mesh=mesh,
        compiler_params=cp,
        scratch_types=(
            [pltpu.VMEM((CHUNK,), jnp.int32)] * 11
            + [pltpu.VMEM((CHUNK, 128), jnp.float32)] * 5
            + [pltpu.VMEM((HROWS,), jnp.float32)] * 2
            + [pltpu.SemaphoreType.DMA]
        ),
    )(_half_body)

    k2 = functools.partial(
        pl.kernel,
        out_type=jax.ShapeDtypeStruct((NW, L), jnp.float32),
        mesh=mesh,
        compiler_params=cp,
        scratch_types=(
            [pltpu.VMEM((HROWS,), jnp.float32)] * 6
            + [pltpu.VMEM((L,), jnp.float32), pltpu.SemaphoreType.DMA]
        ),
    )(_finish_body)

    part = k1(train_r, p_h, p_t, n_h, n_t, eT, rT)
    half = kh(train_r, p_h, p_t, n_h, n_t, packed, packed_r)
    partials = k2(part, half)
    return jnp.sum(partials) / jnp.float32(BATCH_ROWS) + jnp.float32(1.0)


# speedup vs baseline: 2.8969x; 2.8969x over previous
import functools

import jax
import jax.numpy as jnp
from jax import lax
from jax.experimental import pallas as pl
from jax.experimental.pallas import tpu as pltpu
from jax.experimental.pallas import tpu_sc as plsc

NC = 2
NS = 16
L = 16
NW = NC * NS

BATCH_ROWS = 16384
E_ROWS = 1000000
R_ROWS = 1000
DIM = 64
DIMS_PER_SC = DIM // NC
ROWS_PER_TEC = BATCH_ROWS // NS
CHUNK = 128
NCH = ROWS_PER_TEC // CHUNK
CPOS = CHUNK // L


def _sqrt16(x):
    xs = jnp.maximum(x, jnp.float32(1e-30))
    i = lax.bitcast_convert_type(xs, jnp.int32)
    y = lax.bitcast_convert_type(jnp.int32(0x5F3759DF) - (i >> 1), jnp.float32)
    for _ in range(3):
        y = y * (jnp.float32(1.5) - jnp.float32(0.5) * xs * y * y)
    return xs * y


def _accum_body(train_r_hbm, p_h_hbm, p_t_hbm, n_h_hbm, n_t_hbm,
                eT_hbm, rT_hbm, out_hbm,
                ir0, ir1, iph0, iph1, ipt0, ipt1, inh0, inh1, int0, int1,
                wr0, wr1, wph0, wph1, wpt0, wpt1, wnh0, wnh1, wnt0, wnt1,
                sq_p, sq_n,
                rrow0, rrow1, stripe0, stripe1,
                sem_s, sem_r, sem_i, sem_g):
    cid = lax.axis_index("c")
    sid = lax.axis_index("s")
    base = sid * ROWS_PER_TEC
    dim0 = cid * DIMS_PER_SC

    idxb = ((ir0, iph0, ipt0, inh0, int0), (ir1, iph1, ipt1, inh1, int1))
    wb = ((wr0, wph0, wpt0, wnh0, wnt0), (wr1, wph1, wpt1, wnh1, wnt1))
    stripes = (stripe0, stripe1)
    rrows = (rrow0, rrow1)
    src_idx = (train_r_hbm, p_h_hbm, p_t_hbm, n_h_hbm, n_t_hbm)

    zv = jnp.zeros((L,), jnp.float32)

    def zero_body(i, carry):
        sl = pl.ds(i * L, L)
        sq_p[sl] = zv
        sq_n[sl] = zv
        return carry

    lax.fori_loop(0, ROWS_PER_TEC // L, zero_body, jnp.int32(0))

    def fire_idx(cc, par):
        off = base + cc * CHUNK
        for hsrc, dst in zip(src_idx, idxb[par]):
            pltpu.async_copy(hsrc.at[pl.ds(off, CHUNK)], dst, sem_i)

    def drain_idx(par):
        for dst in idxb[par]:
            pltpu.make_async_copy(train_r_hbm.at[pl.ds(0, CHUNK)],
                                  dst, sem_i).wait()

    def fire_gather(par, stripe, rrowb):
        ib = idxb[par]
        ob = wb[par]
        pltpu.async_copy(rrowb.at[ib[0]], ob[0], sem_g)
        for k in range(1, 5):
            pltpu.async_copy(stripe.at[ib[k]], ob[k], sem_g)

    def drain_gather(par):
        for dst in wb[par]:
            pltpu.make_async_copy(out_hbm.at[0, 0, pl.ds(0, CHUNK)],
                                  dst, sem_g).wait()

    def compute(cc, par):
        vr, vph, vpt, vnh, vnt = wb[par]

        def pos_body(i, carry):
            sl = pl.ds(i * L, L)
            gl = pl.ds(cc * CHUNK + i * L, L)
            rv = vr[sl]
            dp = vph[sl] + rv - vpt[sl]
            dn = vnh[sl] + rv - vnt[sl]
            sq_p[gl] = sq_p[gl] + dp * dp
            sq_n[gl] = sq_n[gl] + dn * dn
            return carry

        lax.fori_loop(0, CPOS, pos_body, jnp.int32(0))

    def process(d, sp, rp):
        stripe, rrowb = stripes[sp], rrows[rp]

        def chunk_body(ii, carry):
            for v in range(2):
                cc = ii * 2 + v

                @pl.when(cc >= 2)
                def _():
                    drain_gather(v)
                    compute(cc - 2, v)

                @pl.when(cc < NCH)
                def _():
                    fire_idx(cc, v)

                @pl.when((cc >= 1) & (cc < NCH + 1))
                def _():
                    drain_idx(1 - v)
                    fire_gather(1 - v, stripe, rrowb)
            return carry

        lax.fori_loop(0, NCH // 2 + 1, chunk_body, jnp.int32(0))

    def dim_body(jj, carry):
        for u in range(2):
            j = jj * 2 + u

            @pl.when((j > 0) & (j <= DIMS_PER_SC) & (sid == (j - 1) % NS))
            def _():
                pltpu.make_async_copy(eT_hbm.at[0],
                                      stripes[1 - u], sem_s).wait()

            @pl.when((j > 0) & (j <= DIMS_PER_SC) & (sid == (j + 7) % NS))
            def _():
                pltpu.make_async_copy(rT_hbm.at[0],
                                      rrows[1 - u], sem_r).wait()

            plsc.subcore_barrier()
            d = dim0 + j

            @pl.when((j == 0) & (sid == 0))
            def _():
                pltpu.async_copy(eT_hbm.at[d], stripes[u], sem_s)

            @pl.when((j < DIMS_PER_SC) & (sid == (j + 8) % NS))
            def _():
                pltpu.async_copy(rT_hbm.at[d], rrows[u], sem_r)

            @pl.when((j > 0) & (j <= DIMS_PER_SC))
            def _():
                process(d - 1, 1 - u, 1 - u)

            plsc.subcore_barrier()

            @pl.when((j + 1 < DIMS_PER_SC) & (sid == (j + 1) % NS))
            def _():
                pltpu.async_copy(eT_hbm.at[d + 1], stripes[1 - u], sem_s)
        return carry

    lax.fori_loop(0, DIMS_PER_SC // 2 + 1, dim_body, jnp.int32(0))

    pltpu.sync_copy(sq_p, out_hbm.at[cid, 0, pl.ds(base, ROWS_PER_TEC)])
    pltpu.sync_copy(sq_n, out_hbm.at[cid, 1, pl.ds(base, ROWS_PER_TEC)])


def _finish_body(part_hbm, out_hbm, v0, v1, v2, v3, accbuf, sem):
    cid = lax.axis_index("c")
    sid = lax.axis_index("s")
    wid = sid * NC + cid
    rows = BATCH_ROWS // NW
    base = wid * rows
    d0 = pltpu.async_copy(part_hbm.at[0, 0, pl.ds(base, rows)], v0, sem)
    d1 = pltpu.async_copy(part_hbm.at[1, 0, pl.ds(base, rows)], v1, sem)
    d2 = pltpu.async_copy(part_hbm.at[0, 1, pl.ds(base, rows)], v2, sem)
    d3 = pltpu.async_copy(part_hbm.at[1, 1, pl.ds(base, rows)], v3, sem)
    d0.wait(); d1.wait(); d2.wait(); d3.wait()
    acc = jnp.zeros((L,), jnp.float32)
    for i in range(rows // L):
        sl = pl.ds(i * L, L)
        acc = acc + _sqrt16(v0[sl] + v1[sl]) - _sqrt16(v2[sl] + v3[sl])
    accbuf[...] = acc
    pltpu.sync_copy(accbuf, out_hbm.at[wid])


def kernel(train_r, p_h, p_t, n_h, n_t, e_embed, r_embed):
    mesh = plsc.VectorSubcoreMesh(
        core_axis_name="c", subcore_axis_name="s",
        num_cores=NC, num_subcores=NS)
    cp = pltpu.CompilerParams(needs_layout_passes=False)

    k1 = functools.partial(
        pl.kernel,
        out_type=jax.ShapeDtypeStruct((NC, 2, BATCH_ROWS), jnp.float32),
        mesh=mesh,
        compiler_params=cp,
        scratch_types=(
            [pltpu.VMEM((CHUNK,), jnp.int32)] * 10
            + [pltpu.VMEM((CHUNK,), jnp.float32)] * 10
            + [pltpu.VMEM((ROWS_PER_TEC,), jnp.float32)] * 2
            + [pltpu.MemorySpace.VMEM_SHARED((R_ROWS,), jnp.float32)] * 2
            + [pltpu.MemorySpace.VMEM_SHARED((E_ROWS,), jnp.float32)] * 2
            + [pltpu.SemaphoreType.DMA] * 4
        ),
    )(_accum_body)

    k2 = functools.partial(
        pl.kernel,
        out_type=jax.ShapeDtypeStruct((NW, L), jnp.float32),
        mesh=mesh,
        compiler_params=cp,
        scratch_types=(
            [pltpu.VMEM((BATCH_ROWS // NW,), jnp.float32)] * 4
            + [pltpu.VMEM((L,), jnp.float32), pltpu.SemaphoreType.DMA]
        ),
    )(_finish_body)

    part = k1(train_r, p_h, p_t, n_h, n_t, e_embed.T, r_embed.T)
    partials = k2(part)
    return jnp.sum(partials) / jnp.float32(BATCH_ROWS) + jnp.float32(1.0)
